# Initial kernel scaffold; baseline (speedup 1.0000x reference)
import jax
import jax.numpy as jnp
from jax.experimental import pallas as pl


def kernel(x, rows, cols, w_real, w_imag, phase_angles):
    raise NotImplementedError("write your pallas kernel here")


# speedup vs baseline: 3.3882x; 3.3882x over previous
import functools

import jax
import jax.numpy as jnp
from jax.experimental import pallas as pl
from jax.experimental.pallas import tpu as pltpu

OUT_F = 4096


def _mm_body(x_ref, w_ref, o_ref):
    o_ref[...] = jnp.dot(
        x_ref[0],
        w_ref[0].astype(jnp.bfloat16),
        preferred_element_type=jnp.float32,
    )[None]


@functools.partial(jax.jit, static_argnames=("out_f", "bn", "interpret"))
def _impl(x, rows, cols, w_real, w_imag, phase_angles, out_f, bn=256,
          interpret=False):
    b, s, in_f = x.shape
    cb = jnp.cos(phase_angles)
    vals = w_real[None, :] + cb[:, None] * w_imag[None, :]
    wt = jnp.zeros((b, in_f, out_f), jnp.float32)
    wt = wt.at[:, cols, rows].add(vals)
    xb = x.astype(jnp.bfloat16)

    grid = (b, out_f // bn)
    out = pl.pallas_call(
        _mm_body,
        grid=grid,
        in_specs=[
            pl.BlockSpec((1, s, in_f), lambda i, j: (i, 0, 0)),
            pl.BlockSpec((1, in_f, bn), lambda i, j: (i, 0, j)),
        ],
        out_specs=pl.BlockSpec((1, s, bn), lambda i, j: (i, 0, j)),
        out_shape=jax.ShapeDtypeStruct((b, s, out_f), jnp.float32),
        interpret=interpret,
    )(xb, wt)
    return out


def kernel(x, rows, cols, w_real, w_imag, phase_angles):
    return _impl(x, rows, cols, w_real, w_imag, phase_angles, OUT_F)


# speedup vs baseline: 4.2434x; 1.2524x over previous
import functools

import jax
import jax.numpy as jnp
from jax import lax
from jax.experimental import pallas as pl
from jax.experimental.pallas import tpu as pltpu
from jax.experimental.pallas import tpu_sc as plsc

OUT_F = 4096
IN_F = 4096
TILES = 16
LW = 128
CC = 128
NCB = IN_F // CC
CW = CC * OUT_F
SLICE_W = CW // TILES
ZN = 16384
DUMP = 2048


def _sc_body(ntr, cols_h, rows_h, wr_h, wi_h, c0_h, c1_h, out_h,
             spm, crs, rws, v0, v1, idxc, zbuf, c0v, c1v, dsem):
    c = lax.axis_index("c")
    t = lax.axis_index("s")
    pltpu.sync_copy(cols_h.at[t], crs)
    pltpu.sync_copy(rows_h.at[t], rws)
    pltpu.sync_copy(wr_h.at[t], v0)
    pltpu.sync_copy(wi_h.at[t], v1)
    pltpu.sync_copy(c0_h, c0v)
    pltpu.sync_copy(c1_h, c1v)
    c0 = c0v[...]
    c1 = c1v[...]

    def comb(i, carry):
        for kk in range(LW // 16):
            sl = pl.ds(kk * 16, 16)
            a = v0[i, sl]
            b = v1[i, sl]
            v0[i, sl] = a + c0 * b
            v1[i, sl] = a + c1 * b
        return carry

    lax.fori_loop(0, ntr, comb, 0)

    def zinit(i, carry):
        zbuf[pl.ds(i * 16, 16)] = jnp.zeros((16,), jnp.float32)
        return carry

    lax.fori_loop(0, ZN // 16, zinit, 0)

    iota = lax.iota(jnp.int32, 16)
    for p in range(2 * NCB // 2):
        b = p // (NCB // 2)
        colblk = (p % (NCB // 2)) * 2 + c
        lo = colblk * CC
        k = b * NCB + colblk
        for z in range(SLICE_W // ZN):
            pltpu.sync_copy(zbuf, spm.at[pl.ds(t * SLICE_W + z * ZN, ZN)])

        def phase_idx(i, carry):
            for kk in range(LW // 16):
                sl = pl.ds(kk * 16, 16)
                local = crs[i, sl] - lo
                inb = (local >= 0) & (local < CC)
                sidx = local * OUT_F + rws[i, sl]
                didx = CW + ((i & 15) * LW + kk * 16 + iota)
                idxc[i, sl] = jnp.where(inb, sidx, didx)
            return carry

        lax.fori_loop(0, ntr, phase_idx, 0)
        plsc.subcore_barrier()
        vsrc = v0 if b == 0 else v1

        def scat(i, carry):
            pltpu.async_copy(vsrc.at[i], spm.at[idxc.at[i]], dsem, add=True)
            return carry

        lax.fori_loop(0, ntr, scat, 0)
        pltpu.make_async_copy(wr_h.at[t], vsrc, dsem).wait()
        plsc.subcore_barrier()
        pltpu.sync_copy(spm.at[pl.ds(t * SLICE_W, SLICE_W)],
                        out_h.at[k, pl.ds(t * SLICE_W, SLICE_W)])


def _sc_scatter(cols3, rows3, wr3, wi3, c0, c1):
    ntr = cols3.shape[1]
    mesh = plsc.VectorSubcoreMesh(core_axis_name="c", subcore_axis_name="s")
    fn = pl.kernel(
        functools.partial(_sc_body, ntr),
        out_type=jax.ShapeDtypeStruct((2 * NCB, CW), jnp.float32),
        mesh=mesh,
        scratch_types=[
            pltpu.VMEM_SHARED((CW + DUMP,), jnp.float32),
            pltpu.VMEM((ntr, LW), jnp.int32),
            pltpu.VMEM((ntr, LW), jnp.int32),
            pltpu.VMEM((ntr, LW), jnp.float32),
            pltpu.VMEM((ntr, LW), jnp.float32),
            pltpu.VMEM((ntr, LW), jnp.int32),
            pltpu.VMEM((ZN,), jnp.float32),
            pltpu.VMEM((16,), jnp.float32),
            pltpu.VMEM((16,), jnp.float32),
            pltpu.SemaphoreType.DMA,
        ],
    )
    return fn(cols3, rows3, wr3, wi3, c0, c1)


def _mm_body(x_ref, w_ref, o_ref):
    o_ref[...] = jnp.dot(
        x_ref[0],
        w_ref[0].astype(jnp.bfloat16),
        preferred_element_type=jnp.float32,
    )[None]


@functools.partial(jax.jit, static_argnames=("bn",))
def _impl(x, rows, cols, w_real, w_imag, phase_angles, bn=256):
    b, s, in_f = x.shape
    nnz = rows.shape[0]
    nnz_pad = -(-nnz // (TILES * LW)) * (TILES * LW)
    pad = nnz_pad - nnz
    ntr = nnz_pad // (TILES * LW)
    cols3 = jnp.pad(cols, (0, pad)).reshape(TILES, ntr, LW)
    rows3 = jnp.pad(rows, (0, pad)).reshape(TILES, ntr, LW)
    wr3 = jnp.pad(w_real, (0, pad)).reshape(TILES, ntr, LW)
    wi3 = jnp.pad(w_imag, (0, pad)).reshape(TILES, ntr, LW)
    cb = jnp.cos(phase_angles)
    c0 = jnp.broadcast_to(cb[0], (16,))
    c1 = jnp.broadcast_to(cb[1], (16,))
    wt = _sc_scatter(cols3, rows3, wr3, wi3, c0, c1)
    wt = wt.reshape(b, in_f, OUT_F)
    xb = x.astype(jnp.bfloat16)

    grid = (b, OUT_F // bn)
    out = pl.pallas_call(
        _mm_body,
        grid=grid,
        in_specs=[
            pl.BlockSpec((1, s, in_f), lambda i, j: (i, 0, 0)),
            pl.BlockSpec((1, in_f, bn), lambda i, j: (i, 0, j)),
        ],
        out_specs=pl.BlockSpec((1, s, bn), lambda i, j: (i, 0, j)),
        out_shape=jax.ShapeDtypeStruct((b, s, OUT_F), jnp.float32),
    )(xb, wt)
    return out


def kernel(x, rows, cols, w_real, w_imag, phase_angles):
    return _impl(x, rows, cols, w_real, w_imag, phase_angles)


# speedup vs baseline: 5.7882x; 1.3641x over previous
import functools

import jax
import jax.numpy as jnp
from jax import lax
from jax.experimental import pallas as pl
from jax.experimental.pallas import tpu as pltpu
from jax.experimental.pallas import tpu_sc as plsc

OUT_F = 4096
IN_F = 4096
TILES = 16
LW = 128
CC = 128
NCB = IN_F // CC
CW = CC * OUT_F
SLICE_W = CW // TILES
ZN = 16384
DUMP = 2048


def _sc_body(ntr, cols_h, rows_h, wr_h, wi_h, cb_h, out_h,
             spm, fidx, didx, vv, wib, idxc, zbuf, cbv, dsem):
    c = lax.axis_index("c")
    t = lax.axis_index("s")
    pltpu.sync_copy(cols_h.at[t], fidx)
    pltpu.sync_copy(rows_h.at[t], didx)
    pltpu.sync_copy(wr_h.at[t], vv)
    pltpu.sync_copy(wi_h.at[t], wib)
    pltpu.sync_copy(cb_h, cbv)
    cb = cbv[...]
    iota = lax.iota(jnp.int32, 16)

    def init(i, carry):
        for kk in range(LW // 16):
            sl = pl.ds(kk * 16, 16)
            vv[i, sl] = vv[i, sl] + cb * wib[i, sl]
            fidx[i, sl] = fidx[i, sl] * OUT_F + didx[i, sl]
            didx[i, sl] = CW + ((i & 15) * LW + kk * 16) + iota
        return carry

    lax.fori_loop(0, ntr, init, 0)

    def zinit(i, carry):
        zbuf[pl.ds(i * 16, 16)] = jnp.zeros((16,), jnp.float32)
        return carry

    lax.fori_loop(0, ZN // 16, zinit, 0)

    for p in range(NCB // 2):
        colblk = p * 2 + c
        low = colblk * CW
        for z in range(SLICE_W // ZN):
            zoff = pl.multiple_of(t * SLICE_W + z * ZN, ZN)
            pltpu.sync_copy(zbuf, spm.at[pl.ds(zoff, ZN)])

        def phase_idx(i, carry):
            for kk in range(LW // 16):
                sl = pl.ds(kk * 16, 16)
                local = fidx[i, sl] - low
                inb = (local >= 0) & (local < CW)
                idxc[i, sl] = jnp.where(inb, local, didx[i, sl])
            return carry

        lax.fori_loop(0, ntr, phase_idx, 0)
        plsc.subcore_barrier()

        def scat(i, carry):
            pltpu.async_copy(vv.at[i], spm.at[idxc.at[i]], dsem, add=True)
            return carry

        lax.fori_loop(0, ntr, scat, 0)
        pltpu.make_async_copy(wr_h.at[t], vv, dsem).wait()
        plsc.subcore_barrier()
        pltpu.sync_copy(
            spm.at[pl.ds(pl.multiple_of(t * SLICE_W, SLICE_W), SLICE_W)],
            out_h.at[pl.ds(pl.multiple_of(colblk * CW + t * SLICE_W,
                                          SLICE_W), SLICE_W)])


def _sc_scatter_one(cols3, rows3, wr3, wi3, cvec):
    ntr = cols3.shape[1]
    mesh = plsc.VectorSubcoreMesh(core_axis_name="c", subcore_axis_name="s")
    fn = pl.kernel(
        functools.partial(_sc_body, ntr),
        out_type=jax.ShapeDtypeStruct((NCB * CW,), jnp.float32),
        mesh=mesh,
        scratch_types=[
            pltpu.VMEM_SHARED((CW + DUMP,), jnp.float32),
            pltpu.VMEM((ntr, LW), jnp.int32),
            pltpu.VMEM((ntr, LW), jnp.int32),
            pltpu.VMEM((ntr, LW), jnp.float32),
            pltpu.VMEM((ntr, LW), jnp.float32),
            pltpu.VMEM((ntr, LW), jnp.int32),
            pltpu.VMEM((ZN,), jnp.float32),
            pltpu.VMEM((16,), jnp.float32),
            pltpu.SemaphoreType.DMA,
        ],
    )
    return fn(cols3, rows3, wr3, wi3, cvec)


def _mm_body(x_ref, w_ref, o_ref):
    o_ref[...] = jnp.dot(
        x_ref[...],
        w_ref[...].astype(jnp.bfloat16),
        preferred_element_type=jnp.float32,
    )


def _mm_one(xb, wt, bn):
    s, in_f = xb.shape
    return pl.pallas_call(
        _mm_body,
        grid=(OUT_F // bn,),
        in_specs=[
            pl.BlockSpec((s, in_f), lambda j: (0, 0)),
            pl.BlockSpec((in_f, bn), lambda j: (0, j)),
        ],
        out_specs=pl.BlockSpec((s, bn), lambda j: (0, j)),
        out_shape=jax.ShapeDtypeStruct((s, OUT_F), jnp.float32),
    )(xb, wt)


@functools.partial(jax.jit, static_argnames=("bn",))
def _impl(x, rows, cols, w_real, w_imag, phase_angles, bn=512):
    b, s, in_f = x.shape
    nnz = rows.shape[0]
    nnz_pad = -(-nnz // (TILES * LW)) * (TILES * LW)
    pad = nnz_pad - nnz
    ntr = nnz_pad // (TILES * LW)
    cols3 = jnp.pad(cols, (0, pad)).reshape(TILES, ntr, LW)
    rows3 = jnp.pad(rows, (0, pad)).reshape(TILES, ntr, LW)
    wr3 = jnp.pad(w_real, (0, pad)).reshape(TILES, ntr, LW)
    wi3 = jnp.pad(w_imag, (0, pad)).reshape(TILES, ntr, LW)
    cb = jnp.cos(phase_angles)
    xb = x.astype(jnp.bfloat16)

    outs = []
    for i in range(b):
        ci = jnp.broadcast_to(cb[i], (16,))
        wt = _sc_scatter_one(cols3, rows3, wr3, wi3, ci)
        outs.append(_mm_one(xb[i], wt.reshape(in_f, OUT_F), bn))
    return jnp.stack(outs)


def kernel(x, rows, cols, w_real, w_imag, phase_angles):
    return _impl(x, rows, cols, w_real, w_imag, phase_angles)


# speedup vs baseline: 5.9799x; 1.0331x over previous
import functools

import jax
import jax.numpy as jnp
from jax import lax
from jax.experimental import pallas as pl
from jax.experimental.pallas import tpu as pltpu
from jax.experimental.pallas import tpu_sc as plsc

OUT_F = 4096
IN_F = 4096
TILES = 16
LW = 128
CC = 128
NCB = IN_F // CC
CW = CC * OUT_F
SLICE_W = CW // TILES
ZN = 16384
DUMP = 2048


def _sc_body(ntr, cols_h, rows_h, wr_h, wi_h, cb_h, out_h,
             spm, fidx, didx, vv, wib, idxc, zbuf, cbv, dsem, zsem):
    c = lax.axis_index("c")
    t = lax.axis_index("s")
    pltpu.sync_copy(cols_h.at[t], fidx)
    pltpu.sync_copy(rows_h.at[t], didx)
    pltpu.sync_copy(wr_h.at[t], vv)
    pltpu.sync_copy(wi_h.at[t], wib)
    pltpu.sync_copy(cb_h, cbv)
    cb = cbv[...]
    iota = lax.iota(jnp.int32, 16)

    def init(i, carry):
        for kk in range(LW // 16):
            sl = pl.ds(kk * 16, 16)
            vv[i, sl] = vv[i, sl] + cb * wib[i, sl]
            fidx[i, sl] = fidx[i, sl] * OUT_F + didx[i, sl]
            didx[i, sl] = CW + ((i & 15) * LW + kk * 16) + iota
        return carry

    lax.fori_loop(0, ntr, init, 0)

    def zinit(i, carry):
        zbuf[pl.ds(i * 16, 16)] = jnp.zeros((16,), jnp.float32)
        return carry

    lax.fori_loop(0, ZN // 16, zinit, 0)

    cwu = jnp.uint32(CW)
    for p in range(NCB // 2):
        colblk = p * 2 + c
        low = colblk * CW
        for z in range(SLICE_W // ZN):
            zoff = pl.multiple_of(t * SLICE_W + z * ZN, ZN)
            pltpu.async_copy(zbuf, spm.at[pl.ds(zoff, ZN)], zsem)

        def phase_idx(i, carry):
            for kk in range(LW // 16):
                sl = pl.ds(kk * 16, 16)
                local = fidx[i, sl] - low
                inb = plsc.bitcast(local, jnp.uint32) < cwu
                idxc[i, sl] = jnp.where(inb, local, didx[i, sl])
            return carry

        lax.fori_loop(0, ntr, phase_idx, 0)
        for z in range(SLICE_W // ZN):
            zoff = pl.multiple_of(t * SLICE_W + z * ZN, ZN)
            pltpu.make_async_copy(zbuf, spm.at[pl.ds(zoff, ZN)], zsem).wait()
        plsc.subcore_barrier()

        def scat(i, carry):
            pltpu.async_copy(vv.at[i], spm.at[idxc.at[i]], dsem, add=True)
            return carry

        lax.fori_loop(0, ntr, scat, 0)
        pltpu.make_async_copy(wr_h.at[t], vv, dsem).wait()
        plsc.subcore_barrier()
        pltpu.sync_copy(
            spm.at[pl.ds(pl.multiple_of(t * SLICE_W, SLICE_W), SLICE_W)],
            out_h.at[pl.ds(pl.multiple_of(colblk * CW + t * SLICE_W,
                                          SLICE_W), SLICE_W)])


def _sc_scatter_one(cols3, rows3, wr3, wi3, cvec):
    ntr = cols3.shape[1]
    mesh = plsc.VectorSubcoreMesh(core_axis_name="c", subcore_axis_name="s")
    fn = pl.kernel(
        functools.partial(_sc_body, ntr),
        out_type=jax.ShapeDtypeStruct((NCB * CW,), jnp.float32),
        mesh=mesh,
        scratch_types=[
            pltpu.VMEM_SHARED((CW + DUMP,), jnp.float32),
            pltpu.VMEM((ntr, LW), jnp.int32),
            pltpu.VMEM((ntr, LW), jnp.int32),
            pltpu.VMEM((ntr, LW), jnp.float32),
            pltpu.VMEM((ntr, LW), jnp.float32),
            pltpu.VMEM((ntr, LW), jnp.int32),
            pltpu.VMEM((ZN,), jnp.float32),
            pltpu.VMEM((16,), jnp.float32),
            pltpu.SemaphoreType.DMA,
            pltpu.SemaphoreType.DMA,
        ],
    )
    return fn(cols3, rows3, wr3, wi3, cvec)


def _mm_body(x_ref, w_ref, o_ref):
    o_ref[...] = jnp.dot(
        x_ref[...],
        w_ref[...].astype(jnp.bfloat16),
        preferred_element_type=jnp.float32,
    )


def _mm_one(xb, wt, bn):
    s, in_f = xb.shape
    return pl.pallas_call(
        _mm_body,
        grid=(OUT_F // bn,),
        in_specs=[
            pl.BlockSpec((s, in_f), lambda j: (0, 0)),
            pl.BlockSpec((in_f, bn), lambda j: (0, j)),
        ],
        out_specs=pl.BlockSpec((s, bn), lambda j: (0, j)),
        out_shape=jax.ShapeDtypeStruct((s, OUT_F), jnp.float32),
    )(xb, wt)


@functools.partial(jax.jit, static_argnames=("bn",))
def _impl(x, rows, cols, w_real, w_imag, phase_angles, bn=512):
    b, s, in_f = x.shape
    nnz = rows.shape[0]
    nnz_pad = -(-nnz // (TILES * LW)) * (TILES * LW)
    pad = nnz_pad - nnz
    ntr = nnz_pad // (TILES * LW)
    cols3 = jnp.pad(cols, (0, pad)).reshape(TILES, ntr, LW)
    rows3 = jnp.pad(rows, (0, pad)).reshape(TILES, ntr, LW)
    wr3 = jnp.pad(w_real, (0, pad)).reshape(TILES, ntr, LW)
    wi3 = jnp.pad(w_imag, (0, pad)).reshape(TILES, ntr, LW)
    cb = jnp.cos(phase_angles)
    xb = x.astype(jnp.bfloat16)

    outs = []
    for i in range(b):
        ci = jnp.broadcast_to(cb[i], (16,))
        wt = _sc_scatter_one(cols3, rows3, wr3, wi3, ci)
        outs.append(_mm_one(xb[i], wt.reshape(in_f, OUT_F), bn))
    return jnp.stack(outs)


def kernel(x, rows, cols, w_real, w_imag, phase_angles):
    return _impl(x, rows, cols, w_real, w_imag, phase_angles)
